# baseline (speedup 1.0000x reference)
import jax
import jax.numpy as jnp
from jax import lax
from jax.experimental import pallas as pl
from jax.experimental.pallas import tpu as pltpu

_T = 84
_EPS = 1e-5
_F32 = jnp.float32
_HI = lax.Precision.HIGHEST


def _dot(a, b, precision=None):
    return jnp.dot(a, b, preferred_element_type=_F32, precision=precision)


def _dw_relu(x, wd_ref, bd_ref, *, K, dil):
    C, L = x.shape
    pad = (K - 1) // 2 * dil
    z = jnp.zeros((C, pad), _F32)
    xp = jnp.concatenate([z, x, z], axis=1)
    wd = wd_ref[...]
    acc = jnp.broadcast_to(bd_ref[...], (C, L))
    for k in range(K):
        s = k * dil
        acc = acc + wd[:, k:k + 1] * xp[:, s:s + L]
    return jnp.maximum(acc, 0.0)


def _dw_relu_lc(x, wdt_ref, bdt_ref, *, K):
    L, C = x.shape
    pad = (K - 1) // 2
    z = jnp.zeros((pad, C), _F32)
    xp = jnp.concatenate([z, x, z], axis=0)
    wdt = wdt_ref[...]
    acc = jnp.broadcast_to(bdt_ref[...], (L, C))
    for k in range(K):
        acc = acc + wdt[k:k + 1, :] * xp[k:k + L, :]
    return jnp.maximum(acc, 0.0)


def _ln_ncl(y, gg_ref, bg_ref, s0_ref, s1_ref, s0t_ref, s1t_ref, ts_ref):
    C, L = y.shape
    t = lax.broadcasted_iota(jnp.int32, (C, L), 1)
    first = t < ts_ref[...]
    s0 = s0_ref[...]
    s1 = s1_ref[...]
    inv_c = _F32(1.0 / C)
    a0 = jnp.sum(jnp.where(first, y, 0.0), axis=1, keepdims=True)
    a1 = jnp.sum(y, axis=1, keepdims=True) - a0
    cmu = (_dot(s0t_ref[...], a0, _HI) + _dot(s1t_ref[...], a1, _HI)) * inv_c
    mu0x = _dot(s0, cmu, _HI)
    mu1x = _dot(s1, cmu, _HI)
    d0 = jnp.where(first, y - mu0x, 0.0)
    d1 = jnp.where(first, 0.0, y - mu1x)
    q0 = jnp.sum(d0 * d0, axis=1, keepdims=True)
    q1 = jnp.sum(d1 * d1, axis=1, keepdims=True)
    cvar = (_dot(s0t_ref[...], q0, _HI) + _dot(s1t_ref[...], q1, _HI)) * inv_c
    mu_g = jnp.where(first, _dot(s0, cmu), _dot(s1, cmu))
    var_g = jnp.where(first, _dot(s0, cvar), _dot(s1, cvar))
    return (y - mu_g) * lax.rsqrt(var_g + _EPS) * gg_ref[...] + bg_ref[...]


def _ln_lc(y, ggt_ref, bgt_ref, s0_ref, s1_ref, s0t_ref, s1t_ref, tsr_ref):
    L, C = y.shape
    t = lax.broadcasted_iota(jnp.int32, (L, C), 0)
    first = t < tsr_ref[...]
    s0 = s0_ref[...]
    s1 = s1_ref[...]
    inv_c = _F32(1.0 / C)
    a0 = jnp.sum(jnp.where(first, y, 0.0), axis=0, keepdims=True)
    a1 = jnp.sum(y, axis=0, keepdims=True) - a0
    cmu = (_dot(a0, s0, _HI) + _dot(a1, s1, _HI)) * inv_c
    mu0x = _dot(cmu, s0t_ref[...], _HI)
    mu1x = _dot(cmu, s1t_ref[...], _HI)
    d0 = jnp.where(first, y - mu0x, 0.0)
    d1 = jnp.where(first, 0.0, y - mu1x)
    q0 = jnp.sum(d0 * d0, axis=0, keepdims=True)
    q1 = jnp.sum(d1 * d1, axis=0, keepdims=True)
    cvar = (_dot(q0, s0, _HI) + _dot(q1, s1, _HI)) * inv_c
    mu_g = jnp.where(first, _dot(cmu, s0t_ref[...]), _dot(cmu, s1t_ref[...]))
    var_g = jnp.where(first, _dot(cvar, s0t_ref[...]), _dot(cvar, s1t_ref[...]))
    return (y - mu_g) * lax.rsqrt(var_g + _EPS) * ggt_ref[...] + bgt_ref[...]


def _fused_kernel(feat_ref, pv_ref,
                  w1d, b1d, w1p, b1p, g1, be1, s01, s11, s01t, s11t, ts1,
                  w2d, b2d, w2p, b2p, g2, be2, s02, s12, s02t, s12t, ts2,
                  w3d, b3d, w3p, b3p, g3, be3, s03, s13, s03t, s13t, ts3,
                  wpdt, bpdt, wppt, bppt, gpt, bept, s0p, s1p, s0pt, s1pt, tsp,
                  w1a, w1b, fb1, fw2, fb2, fw3, fb3,
                  o_ref):
    x = feat_ref[...]
    h = _dw_relu(x, w1d, b1d, K=11, dil=1)
    y1 = jnp.maximum(
        jnp.dot(w1p[...], h, preferred_element_type=_F32) + b1p[...], 0.0)
    z1 = _ln_ncl(y1, g1, be1, s01, s11, s01t, s11t, ts1)
    h = _dw_relu(z1, w2d, b2d, K=11, dil=2)
    y2 = jnp.maximum(
        jnp.dot(w2p[...], h, preferred_element_type=_F32) + b2p[...], 0.0)
    z2 = _ln_ncl(y2, g2, be2, s02, s12, s02t, s12t, ts2)
    h = _dw_relu(z2, w3d, b3d, K=11, dil=4)
    y3 = jnp.maximum(
        jnp.dot(w3p[...], h, preferred_element_type=_F32) + b3p[...], 0.0)
    z3 = _ln_ncl(y3, g3, be3, s03, s13, s03t, s13t, ts3)
    p3 = jnp.mean(z3, axis=1, keepdims=True)

    xp = pv_ref[...]
    hp = _dw_relu_lc(xp, wpdt, bpdt, K=11)
    yp = jnp.maximum(
        jnp.dot(hp, wppt[...], preferred_element_type=_F32) + bppt[...], 0.0)
    zp = _ln_lc(yp, gpt, bept, s0p, s1p, s0pt, s1pt, tsp)

    pp = jnp.mean(zp, axis=0, keepdims=True)
    xb = lax.dot_general(w1b[...], pp, (((1,), (1,)), ((), ())),
                         preferred_element_type=_F32)
    h1 = _dot(w1a[...], p3) + xb + fb1[...]
    h2 = _dot(fw2[...], h1) + fb2[...]
    o_ref[...] = _dot(fw3[...], h2) + fb3[...]


def _w2d(shape):
    return pl.BlockSpec(tuple(shape), lambda i: (0, 0))


def kernel(feat, pv,
           w1d, b1d, w1p, b1p,
           w2d, b2d, w2p, b2p,
           w3d, b3d, w3p, b3p,
           wpd, bpd, wpp, bpp,
           ln1_g_grid, ln1_b_grid, ln1_s0, ln1_s1, ln1_tstar,
           ln2_g_grid, ln2_b_grid, ln2_s0, ln2_s1, ln2_tstar,
           ln3_g_grid, ln3_b_grid, ln3_s0, ln3_s1, ln3_tstar,
           lnp_g_grid, lnp_b_grid, lnp_s0, lnp_s1, lnp_tstar,
           fc_w1a, fc_w1b, fc_b1, fc_w1s, fc_b1s,
           fc_w2, fc_b2, fc_w3, fc_b3):
    B = feat.shape[0]
    pvr = pv.reshape(B, _T, 16 * 54)

    operands = [
        feat, pvr,
        w1d, b1d, w1p, b1p,
        ln1_g_grid, ln1_b_grid, ln1_s0, ln1_s1, ln1_s0.T, ln1_s1.T, ln1_tstar,
        w2d, b2d, w2p, b2p,
        ln2_g_grid, ln2_b_grid, ln2_s0, ln2_s1, ln2_s0.T, ln2_s1.T, ln2_tstar,
        w3d, b3d, w3p, b3p,
        ln3_g_grid, ln3_b_grid, ln3_s0, ln3_s1, ln3_s0.T, ln3_s1.T, ln3_tstar,
        wpd.T, bpd.reshape(1, -1), wpp.T, bpp.reshape(1, -1),
        lnp_g_grid.T, lnp_b_grid.T, lnp_s0, lnp_s1, lnp_s0.T, lnp_s1.T,
        lnp_tstar.reshape(1, -1),
        fc_w1a, fc_w1b, fc_b1, fc_w2, fc_b2, fc_w3, fc_b3,
    ]
    in_specs = (
        [pl.BlockSpec((None, 769, _T), lambda i: (i, 0, 0)),
         pl.BlockSpec((None, _T, 864), lambda i: (i, 0, 0))]
        + [_w2d(op.shape) for op in operands[2:]]
    )
    out = pl.pallas_call(
        _fused_kernel,
        out_shape=jax.ShapeDtypeStruct((B, 1, 1), _F32),
        grid=(B,),
        in_specs=in_specs,
        out_specs=pl.BlockSpec((None, 1, 1), lambda i: (i, 0, 0)),
        compiler_params=pltpu.CompilerParams(
            dimension_semantics=("parallel",)),
    )(*operands)
    return jnp.squeeze(out)


# speedup vs baseline: 2.5376x; 2.5376x over previous
import jax
import jax.numpy as jnp
from jax import lax
from jax.experimental import pallas as pl
from jax.experimental.pallas import tpu as pltpu

_T = 84
_EPS = 1e-5
_F32 = jnp.float32
_BF16 = jnp.bfloat16


def _dot(a, b):
    return jnp.dot(a, b, preferred_element_type=_F32)


def _dotr(a, b):
    return lax.dot_general(a, b, (((1,), (1,)), ((), ())),
                           preferred_element_type=_F32)


def _dw_relu_lc(x, wdt_ref, bdt_ref, *, K, dil):
    L, C = x.shape
    pad = (K - 1) // 2 * dil
    z = jnp.zeros((pad, C), _F32)
    xp = jnp.concatenate([z, x, z], axis=0)
    wdt = wdt_ref[...]
    acc = jnp.broadcast_to(bdt_ref[...], (L, C))
    for k in range(K):
        s = k * dil
        acc = acc + wdt[k:k + 1, :] * xp[s:s + L, :]
    return jnp.maximum(acc, 0.0)


def _split3(r):
    h0 = r.astype(_BF16).astype(_F32)
    r1 = r - h0
    h1 = r1.astype(_BF16).astype(_F32)
    return jnp.concatenate([h0, h1, r1 - h1], axis=0)


def _ln_lc(y, gt_ref, bt_ref, tsr_ref, scat_ref, s0t_ref, s1t_ref):
    L, C = y.shape
    t = lax.broadcasted_iota(jnp.int32, (L, C), 0)
    first = t < tsr_ref[...]
    inv_c = _F32(1.0 / C)
    scat = scat_ref[...]
    s0t = s0t_ref[...]
    s1t = s1t_ref[...]

    def chunk_stat(r0, r1):
        rows = jnp.concatenate([_split3(r0), _split3(r1)], axis=1)
        s = _dot(rows, scat)
        return jnp.sum(s, axis=0, keepdims=True) * inv_c

    a0 = jnp.sum(jnp.where(first, y, 0.0), axis=0, keepdims=True)
    a1 = jnp.sum(y, axis=0, keepdims=True) - a0
    cmu = chunk_stat(a0, a1)
    cmu3 = _split3(cmu)
    mu0x = jnp.sum(_dot(cmu3, s0t), axis=0, keepdims=True)
    mu1x = jnp.sum(_dot(cmu3, s1t), axis=0, keepdims=True)
    d0 = jnp.where(first, y - mu0x, 0.0)
    d1 = jnp.where(first, 0.0, y - mu1x)
    q0 = jnp.sum(d0 * d0, axis=0, keepdims=True)
    q1 = jnp.sum(d1 * d1, axis=0, keepdims=True)
    cvar = chunk_stat(q0, q1)
    mv = jnp.concatenate([cmu3[0:1, :],
                          cvar.astype(_BF16).astype(_F32)], axis=0)
    Q0 = _dot(mv, s0t)
    Q1 = _dot(mv, s1t)
    mu_g = jnp.where(first, Q0[0:1, :], Q1[0:1, :])
    var_g = jnp.where(first, Q0[1:2, :], Q1[1:2, :])
    return (y - mu_g) * lax.rsqrt(var_g + _EPS) * gt_ref[...] + bt_ref[...]


def _fused_kernel(feat_ref, pv_ref,
                  wd1, bd1, wp1, bq1, g1, bb1, ts1, sc1, s0t1, s1t1,
                  wd2, bd2, wp2, bq2, g2, bb2, ts2, sc2, s0t2, s1t2,
                  wd3, bd3, wp3, bq3, g3, bb3, ts3, sc3, s0t3, s1t3,
                  wdp, bdp, wpp, bqp, gp, bbp, tsp, scp, s0tp, s1tp,
                  w1a, w1b, fb1, fw2, fb2, fw3, fb3,
                  o_ref):
    x = feat_ref[...]
    h = _dw_relu_lc(x, wd1, bd1, K=11, dil=1)
    y1 = jnp.maximum(_dot(h, wp1[...]) + bq1[...], 0.0)
    z1 = _ln_lc(y1, g1, bb1, ts1, sc1, s0t1, s1t1)
    h = _dw_relu_lc(z1, wd2, bd2, K=11, dil=2)
    y2 = jnp.maximum(_dot(h, wp2[...]) + bq2[...], 0.0)
    z2 = _ln_lc(y2, g2, bb2, ts2, sc2, s0t2, s1t2)
    h = _dw_relu_lc(z2, wd3, bd3, K=11, dil=4)
    y3 = jnp.maximum(_dot(h, wp3[...]) + bq3[...], 0.0)
    z3 = _ln_lc(y3, g3, bb3, ts3, sc3, s0t3, s1t3)
    p3 = jnp.mean(z3, axis=0, keepdims=True)

    xp = pv_ref[...]
    hp = _dw_relu_lc(xp, wdp, bdp, K=11, dil=1)
    yp = jnp.maximum(_dot(hp, wpp[...]) + bqp[...], 0.0)
    zp = _ln_lc(yp, gp, bbp, tsp, scp, s0tp, s1tp)
    pp = jnp.mean(zp, axis=0, keepdims=True)

    h1 = _dotr(p3, w1a[...]) + _dotr(pp, w1b[...]) + fb1[...]
    h2 = _dotr(h1, fw2[...]) + fb2[...]
    prod = (h2.astype(_BF16).astype(_F32)
            * fw3[...].astype(_BF16).astype(_F32))
    o_ref[...] = jnp.sum(prod, axis=1, keepdims=True) + fb3[...]


def _w2d(shape):
    return pl.BlockSpec(tuple(shape), lambda i: (0, 0))


def _ln_pack(g_grid, b_grid, s0, s1, tstar):
    C = g_grid.shape[0]
    return [g_grid.T, b_grid.T, tstar.reshape(1, C),
            jnp.concatenate([s0, s1], axis=0), s0.T, s1.T]


def kernel(feat, pv,
           w1d, b1d, w1p, b1p,
           w2d, b2d, w2p, b2p,
           w3d, b3d, w3p, b3p,
           wpd, bpd, wpp, bpp,
           ln1_g_grid, ln1_b_grid, ln1_s0, ln1_s1, ln1_tstar,
           ln2_g_grid, ln2_b_grid, ln2_s0, ln2_s1, ln2_tstar,
           ln3_g_grid, ln3_b_grid, ln3_s0, ln3_s1, ln3_tstar,
           lnp_g_grid, lnp_b_grid, lnp_s0, lnp_s1, lnp_tstar,
           fc_w1a, fc_w1b, fc_b1, fc_w1s, fc_b1s,
           fc_w2, fc_b2, fc_w3, fc_b3):
    B = feat.shape[0]
    feat_t = jnp.swapaxes(feat, 1, 2)
    pvr = pv.reshape(B, _T, 16 * 54)

    row = lambda v: v.reshape(1, -1)
    operands = [
        feat_t, pvr,
        w1d.T, row(b1d), w1p.T, row(b1p),
        *_ln_pack(ln1_g_grid, ln1_b_grid, ln1_s0, ln1_s1, ln1_tstar),
        w2d.T, row(b2d), w2p.T, row(b2p),
        *_ln_pack(ln2_g_grid, ln2_b_grid, ln2_s0, ln2_s1, ln2_tstar),
        w3d.T, row(b3d), w3p.T, row(b3p),
        *_ln_pack(ln3_g_grid, ln3_b_grid, ln3_s0, ln3_s1, ln3_tstar),
        wpd.T, row(bpd), wpp.T, row(bpp),
        *_ln_pack(lnp_g_grid, lnp_b_grid, lnp_s0, lnp_s1, lnp_tstar),
        fc_w1a, fc_w1b, row(fc_b1), fc_w2, row(fc_b2), fc_w3, fc_b3,
    ]
    in_specs = (
        [pl.BlockSpec((None, _T, 769), lambda i: (i, 0, 0)),
         pl.BlockSpec((None, _T, 864), lambda i: (i, 0, 0))]
        + [_w2d(op.shape) for op in operands[2:]]
    )
    out = pl.pallas_call(
        _fused_kernel,
        out_shape=jax.ShapeDtypeStruct((B, 1, 1), _F32),
        grid=(B,),
        in_specs=in_specs,
        out_specs=pl.BlockSpec((None, 1, 1), lambda i: (i, 0, 0)),
        compiler_params=pltpu.CompilerParams(
            dimension_semantics=("parallel",)),
    )(*operands)
    return jnp.squeeze(out)


# speedup vs baseline: 2.5656x; 1.0110x over previous
import jax
import jax.numpy as jnp
from jax import lax
from jax.experimental import pallas as pl
from jax.experimental.pallas import tpu as pltpu

_T = 84
_EPS = 1e-5
_F32 = jnp.float32
_BF16 = jnp.bfloat16


def _dot(a, b):
    return jnp.dot(a, b, preferred_element_type=_F32)


def _dotr(a, b):
    return lax.dot_general(a, b, (((1,), (1,)), ((), ())),
                           preferred_element_type=_F32)


def _tree_sum(terms):
    while len(terms) > 1:
        nxt = [terms[i] + terms[i + 1] for i in range(0, len(terms) - 1, 2)]
        if len(terms) % 2:
            nxt.append(terms[-1])
        terms = nxt
    return terms[0]


def _dw_relu_lc(x, wdt_ref, bdt_ref, *, K, dil):
    L, C = x.shape
    pad = (K - 1) // 2 * dil
    z = jnp.zeros((pad, C), _F32)
    xp = jnp.concatenate([z, x, z], axis=0)
    wdt = wdt_ref[...]
    terms = [jnp.broadcast_to(bdt_ref[...], (L, C))]
    for k in range(K):
        s = k * dil
        terms.append(wdt[k:k + 1, :] * xp[s:s + L, :])
    return jnp.maximum(_tree_sum(terms), 0.0)


def _split3(r):
    h0 = r.astype(_BF16).astype(_F32)
    r1 = r - h0
    h1 = r1.astype(_BF16).astype(_F32)
    return jnp.concatenate([h0, h1, r1 - h1], axis=0)


def _ln_lc(y, gt_ref, bt_ref, tsr_ref, scat_ref, s0t_ref, s1t_ref):
    L, C = y.shape
    t = lax.broadcasted_iota(jnp.int32, (L, C), 0)
    first = t < tsr_ref[...]
    inv_c = _F32(1.0 / C)
    scat = scat_ref[...]
    s0t = s0t_ref[...]
    s1t = s1t_ref[...]

    def chunk_stat(r0, r1):
        rows = jnp.concatenate([_split3(r0), _split3(r1)], axis=1)
        s = _dot(rows, scat)
        return jnp.sum(s, axis=0, keepdims=True) * inv_c

    a0 = jnp.sum(jnp.where(first, y, 0.0), axis=0, keepdims=True)
    a1 = jnp.sum(y, axis=0, keepdims=True) - a0
    cmu = chunk_stat(a0, a1)
    cmu3 = _split3(cmu)
    mu0x = jnp.sum(_dot(cmu3, s0t), axis=0, keepdims=True)
    mu1x = jnp.sum(_dot(cmu3, s1t), axis=0, keepdims=True)
    d0 = jnp.where(first, y - mu0x, 0.0)
    d1 = jnp.where(first, 0.0, y - mu1x)
    q0 = jnp.sum(d0 * d0, axis=0, keepdims=True)
    q1 = jnp.sum(d1 * d1, axis=0, keepdims=True)
    cvar = chunk_stat(q0, q1)
    mv = jnp.concatenate([cmu3[0:1, :],
                          cvar.astype(_BF16).astype(_F32)], axis=0)
    Q0 = _dot(mv, s0t)
    Q1 = _dot(mv, s1t)
    mu_g = jnp.where(first, Q0[0:1, :], Q1[0:1, :])
    var_g = jnp.where(first, Q0[1:2, :], Q1[1:2, :])
    return (y - mu_g) * lax.rsqrt(var_g + _EPS) * gt_ref[...] + bt_ref[...]


_NR = 2


def _row_forward(x, xp,
                 wd1, bd1, wp1, bq1, g1, bb1, ts1, sc1, s0t1, s1t1,
                 wd2, bd2, wp2, bq2, g2, bb2, ts2, sc2, s0t2, s1t2,
                 wd3, bd3, wp3, bq3, g3, bb3, ts3, sc3, s0t3, s1t3,
                 wdp, bdp, wpp, bqp, gp, bbp, tsp, scp, s0tp, s1tp,
                 w1a, w1b, fb1, fw2, fb2, fw3, fb3):
    h = _dw_relu_lc(x, wd1, bd1, K=11, dil=1)
    y1 = jnp.maximum(_dot(h, wp1[...]) + bq1[...], 0.0)
    z1 = _ln_lc(y1, g1, bb1, ts1, sc1, s0t1, s1t1)
    h = _dw_relu_lc(z1, wd2, bd2, K=11, dil=2)
    y2 = jnp.maximum(_dot(h, wp2[...]) + bq2[...], 0.0)
    z2 = _ln_lc(y2, g2, bb2, ts2, sc2, s0t2, s1t2)
    h = _dw_relu_lc(z2, wd3, bd3, K=11, dil=4)
    y3 = jnp.maximum(_dot(h, wp3[...]) + bq3[...], 0.0)
    z3 = _ln_lc(y3, g3, bb3, ts3, sc3, s0t3, s1t3)
    p3 = jnp.mean(z3, axis=0, keepdims=True)

    hp = _dw_relu_lc(xp, wdp, bdp, K=11, dil=1)
    yp = jnp.maximum(_dot(hp, wpp[...]) + bqp[...], 0.0)
    zp = _ln_lc(yp, gp, bbp, tsp, scp, s0tp, s1tp)
    pp = jnp.mean(zp, axis=0, keepdims=True)

    h1 = _dotr(p3, w1a[...]) + _dotr(pp, w1b[...]) + fb1[...]
    h2 = _dotr(h1, fw2[...]) + fb2[...]
    prod = (h2.astype(_BF16).astype(_F32)
            * fw3[...].astype(_BF16).astype(_F32))
    return jnp.sum(prod, axis=1, keepdims=True) + fb3[...]


def _fused_kernel(feat_ref, pv_ref, *args):
    wargs, o_ref = args[:-1], args[-1]
    for r in range(_NR):
        o_ref[r:r + 1, :] = _row_forward(feat_ref[r], pv_ref[r], *wargs)


def _w2d(shape):
    return pl.BlockSpec(tuple(shape), lambda i: (0, 0))


def _ln_pack(g_grid, b_grid, s0, s1, tstar):
    C = g_grid.shape[0]
    return [g_grid.T, b_grid.T, tstar.reshape(1, C),
            jnp.concatenate([s0, s1], axis=0), s0.T, s1.T]


def kernel(feat, pv,
           w1d, b1d, w1p, b1p,
           w2d, b2d, w2p, b2p,
           w3d, b3d, w3p, b3p,
           wpd, bpd, wpp, bpp,
           ln1_g_grid, ln1_b_grid, ln1_s0, ln1_s1, ln1_tstar,
           ln2_g_grid, ln2_b_grid, ln2_s0, ln2_s1, ln2_tstar,
           ln3_g_grid, ln3_b_grid, ln3_s0, ln3_s1, ln3_tstar,
           lnp_g_grid, lnp_b_grid, lnp_s0, lnp_s1, lnp_tstar,
           fc_w1a, fc_w1b, fc_b1, fc_w1s, fc_b1s,
           fc_w2, fc_b2, fc_w3, fc_b3):
    B = feat.shape[0]
    feat_t = jnp.swapaxes(feat, 1, 2)
    pvr = pv.reshape(B, _T, 16 * 54)

    row = lambda v: v.reshape(1, -1)
    operands = [
        feat_t, pvr,
        w1d.T, row(b1d), w1p.T, row(b1p),
        *_ln_pack(ln1_g_grid, ln1_b_grid, ln1_s0, ln1_s1, ln1_tstar),
        w2d.T, row(b2d), w2p.T, row(b2p),
        *_ln_pack(ln2_g_grid, ln2_b_grid, ln2_s0, ln2_s1, ln2_tstar),
        w3d.T, row(b3d), w3p.T, row(b3p),
        *_ln_pack(ln3_g_grid, ln3_b_grid, ln3_s0, ln3_s1, ln3_tstar),
        wpd.T, row(bpd), wpp.T, row(bpp),
        *_ln_pack(lnp_g_grid, lnp_b_grid, lnp_s0, lnp_s1, lnp_tstar),
        fc_w1a, fc_w1b, row(fc_b1), fc_w2, row(fc_b2), fc_w3, fc_b3,
    ]
    in_specs = (
        [pl.BlockSpec((_NR, _T, 769), lambda i: (i, 0, 0)),
         pl.BlockSpec((_NR, _T, 864), lambda i: (i, 0, 0))]
        + [_w2d(op.shape) for op in operands[2:]]
    )
    out = pl.pallas_call(
        _fused_kernel,
        out_shape=jax.ShapeDtypeStruct((B // _NR, _NR, 1), _F32),
        grid=(B // _NR,),
        in_specs=in_specs,
        out_specs=pl.BlockSpec((None, _NR, 1), lambda i: (i, 0, 0)),
        compiler_params=pltpu.CompilerParams(
            dimension_semantics=("parallel",)),
    )(*operands)
    return out.reshape(B)


# speedup vs baseline: 3.3977x; 1.3243x over previous
import jax
import jax.numpy as jnp
from jax import lax
from jax.experimental import pallas as pl
from jax.experimental.pallas import tpu as pltpu

_T = 84
_EPS = 1e-5
_F32 = jnp.float32
_BF16 = jnp.bfloat16


def _dot(a, b):
    return jnp.dot(a, b, preferred_element_type=_F32)


def _dotr(a, b):
    return lax.dot_general(a, b, (((1,), (1,)), ((), ())),
                           preferred_element_type=_F32)


def _tree_sum(terms):
    while len(terms) > 1:
        nxt = [terms[i] + terms[i + 1] for i in range(0, len(terms) - 1, 2)]
        if len(terms) % 2:
            nxt.append(terms[-1])
        terms = nxt
    return terms[0]


_TOP = 24
_STR = 112
_W = 196


def _dw_relu_st(xs, wdt_ref, bdt_ref, *, K, dil):
    C = xs.shape[1]
    pad = (K - 1) // 2 * dil
    wdt = wdt_ref[...]
    terms = [jnp.broadcast_to(bdt_ref[...], (_W, C))]
    for k in range(K):
        o = _TOP + k * dil - pad
        terms.append(wdt[k:k + 1, :] * xs[o:o + _W, :])
    return jnp.maximum(_tree_sum(terms), 0.0)


def _stack_pad(x0, x1):
    C = x0.shape[1]
    return jnp.concatenate(
        [jnp.zeros((_TOP, C), _F32), x0, jnp.zeros((28, C), _F32),
         x1, jnp.zeros((20, C), _F32)], axis=0)


def _repad(z):
    C = z.shape[1]
    return jnp.concatenate(
        [jnp.zeros((_TOP, C), _F32), z, jnp.zeros((20, C), _F32)], axis=0)


def _split3(r):
    h0 = r.astype(_BF16).astype(_F32)
    r1 = r - h0
    h1 = r1.astype(_BF16).astype(_F32)
    return jnp.concatenate([h0, h1, r1 - h1], axis=0)


def _ln_pair(y, gt_ref, bt_ref, tsr_ref, scat_ref, s0t_ref, s1t_ref):
    C = y.shape[1]
    y0 = y[0:84, :]
    y1 = y[112:196, :]
    t = lax.broadcasted_iota(jnp.int32, (84, C), 0)
    first = t < tsr_ref[...]
    inv_c = _F32(1.0 / C)
    scat = scat_ref[...]
    s0t = s0t_ref[...]
    s1t = s1t_ref[...]

    def stat_rows(r):
        a0 = jnp.sum(jnp.where(first, r, 0.0), axis=0, keepdims=True)
        a1 = jnp.sum(r, axis=0, keepdims=True) - a0
        return jnp.concatenate([_split3(a0), _split3(a1)], axis=1)

    st = _dot(jnp.concatenate([stat_rows(y0), stat_rows(y1)], axis=0), scat)
    cmu0 = jnp.sum(st[0:3, :], axis=0, keepdims=True) * inv_c
    cmu1 = jnp.sum(st[3:6, :], axis=0, keepdims=True) * inv_c
    cpair = jnp.concatenate([_split3(cmu0), _split3(cmu1)], axis=0)
    X0 = _dot(cpair, s0t)
    X1 = _dot(cpair, s1t)
    mu0x_0 = jnp.sum(X0[0:3, :], axis=0, keepdims=True)
    mu1x_0 = jnp.sum(X1[0:3, :], axis=0, keepdims=True)
    mu0x_1 = jnp.sum(X0[3:6, :], axis=0, keepdims=True)
    mu1x_1 = jnp.sum(X1[3:6, :], axis=0, keepdims=True)

    def var_rows(r, m0, m1):
        d0 = jnp.where(first, r - m0, 0.0)
        d1 = jnp.where(first, 0.0, r - m1)
        q0 = jnp.sum(d0 * d0, axis=0, keepdims=True)
        q1 = jnp.sum(d1 * d1, axis=0, keepdims=True)
        return jnp.concatenate([_split3(q0), _split3(q1)], axis=1)

    sv = _dot(jnp.concatenate([var_rows(y0, mu0x_0, mu1x_0),
                               var_rows(y1, mu0x_1, mu1x_1)], axis=0), scat)
    cvar0 = jnp.sum(sv[0:3, :], axis=0, keepdims=True) * inv_c
    cvar1 = jnp.sum(sv[3:6, :], axis=0, keepdims=True) * inv_c
    bq = lambda v: v.astype(_BF16).astype(_F32)
    mv = jnp.concatenate([bq(cmu0), bq(cvar0), bq(cmu1), bq(cvar1)], axis=0)
    Q0 = _dot(mv, s0t)
    Q1 = _dot(mv, s1t)
    g = gt_ref[...]
    b = bt_ref[...]

    def norm(r, i):
        mu_g = jnp.where(first, Q0[2 * i:2 * i + 1, :], Q1[2 * i:2 * i + 1, :])
        var_g = jnp.where(first, Q0[2 * i + 1:2 * i + 2, :],
                          Q1[2 * i + 1:2 * i + 2, :])
        return (r - mu_g) * lax.rsqrt(var_g + _EPS) * g + b

    return jnp.concatenate(
        [norm(y0, 0), jnp.zeros((28, C), _F32), norm(y1, 1)], axis=0)


_NR = 2


def _pair_forward(feat_ref, pv_ref,
                  wd1, bd1, wp1, bq1, g1, bb1, ts1, sc1, s0t1, s1t1,
                  wd2, bd2, wp2, bq2, g2, bb2, ts2, sc2, s0t2, s1t2,
                  wd3, bd3, wp3, bq3, g3, bb3, ts3, sc3, s0t3, s1t3,
                  wdp, bdp, wpp, bqp, gp, bbp, tsp, scp, s0tp, s1tp,
                  w1a, w1b, fb1, fw2, fb2, fw3, fb3):
    xs = _stack_pad(feat_ref[0], feat_ref[1])
    h = _dw_relu_st(xs, wd1, bd1, K=11, dil=1)
    y1 = jnp.maximum(_dot(h, wp1[...]) + bq1[...], 0.0)
    z1 = _ln_pair(y1, g1, bb1, ts1, sc1, s0t1, s1t1)
    h = _dw_relu_st(_repad(z1), wd2, bd2, K=11, dil=2)
    y2 = jnp.maximum(_dot(h, wp2[...]) + bq2[...], 0.0)
    z2 = _ln_pair(y2, g2, bb2, ts2, sc2, s0t2, s1t2)
    h = _dw_relu_st(_repad(z2), wd3, bd3, K=11, dil=4)
    y3 = jnp.maximum(_dot(h, wp3[...]) + bq3[...], 0.0)
    z3 = _ln_pair(y3, g3, bb3, ts3, sc3, s0t3, s1t3)

    xsp = _stack_pad(pv_ref[0], pv_ref[1])
    hp = _dw_relu_st(xsp, wdp, bdp, K=11, dil=1)
    yp = jnp.maximum(_dot(hp, wpp[...]) + bqp[...], 0.0)
    zp = _ln_pair(yp, gp, bbp, tsp, scp, s0tp, s1tp)

    mrow = lambda z: jnp.concatenate(
        [jnp.mean(z[0:84, :], axis=0, keepdims=True),
         jnp.mean(z[112:196, :], axis=0, keepdims=True)], axis=0)
    p3 = mrow(z3)
    pp = mrow(zp)
    h1 = _dotr(p3, w1a[...]) + _dotr(pp, w1b[...]) + fb1[...]
    h2 = _dotr(h1, fw2[...]) + fb2[...]
    prod = (h2.astype(_BF16).astype(_F32)
            * fw3[...].astype(_BF16).astype(_F32))
    return jnp.sum(prod, axis=1, keepdims=True) + fb3[...]


def _fused_kernel(feat_ref, pv_ref, *args):
    wargs, o_ref = args[:-1], args[-1]
    o_ref[...] = _pair_forward(feat_ref, pv_ref, *wargs)


def _w2d(shape):
    n = len(shape)
    return pl.BlockSpec(tuple(shape), lambda i, n=n: (0,) * n)


def _ln_pack(g_grid, b_grid, s0, s1, tstar):
    C = g_grid.shape[0]
    return [g_grid.T, b_grid.T, tstar.reshape(1, C),
            jnp.concatenate([s0, s1], axis=0), s0.T, s1.T]


def kernel(feat, pv,
           w1d, b1d, w1p, b1p,
           w2d, b2d, w2p, b2p,
           w3d, b3d, w3p, b3p,
           wpd, bpd, wpp, bpp,
           ln1_g_grid, ln1_b_grid, ln1_s0, ln1_s1, ln1_tstar,
           ln2_g_grid, ln2_b_grid, ln2_s0, ln2_s1, ln2_tstar,
           ln3_g_grid, ln3_b_grid, ln3_s0, ln3_s1, ln3_tstar,
           lnp_g_grid, lnp_b_grid, lnp_s0, lnp_s1, lnp_tstar,
           fc_w1a, fc_w1b, fc_b1, fc_w1s, fc_b1s,
           fc_w2, fc_b2, fc_w3, fc_b3):
    B = feat.shape[0]
    feat_t = jnp.swapaxes(feat, 1, 2)
    pvr = pv.reshape(B, _T, 16 * 54)

    row = lambda v: v.reshape(1, -1)
    operands = [
        feat_t, pvr,
        w1d.T, row(b1d), w1p.T, row(b1p),
        *_ln_pack(ln1_g_grid, ln1_b_grid, ln1_s0, ln1_s1, ln1_tstar),
        w2d.T, row(b2d), w2p.T, row(b2p),
        *_ln_pack(ln2_g_grid, ln2_b_grid, ln2_s0, ln2_s1, ln2_tstar),
        w3d.T, row(b3d), w3p.T, row(b3p),
        *_ln_pack(ln3_g_grid, ln3_b_grid, ln3_s0, ln3_s1, ln3_tstar),
        wpd.T, row(bpd), wpp.T, row(bpp),
        *_ln_pack(lnp_g_grid, lnp_b_grid, lnp_s0, lnp_s1, lnp_tstar),
        fc_w1a, fc_w1b, row(fc_b1), fc_w2, row(fc_b2), fc_w3, fc_b3,
    ]
    in_specs = (
        [pl.BlockSpec((_NR, _T, 769), lambda i: (i, 0, 0)),
         pl.BlockSpec((_NR, _T, 864), lambda i: (i, 0, 0))]
        + [_w2d(op.shape) for op in operands[2:]]
    )
    out = pl.pallas_call(
        _fused_kernel,
        out_shape=jax.ShapeDtypeStruct((B // _NR, _NR, 1), _F32),
        grid=(B // _NR,),
        in_specs=in_specs,
        out_specs=pl.BlockSpec((None, _NR, 1), lambda i: (i, 0, 0)),
        compiler_params=pltpu.CompilerParams(
            dimension_semantics=("parallel",)),
    )(*operands)
    return out.reshape(B)


# speedup vs baseline: 4.1012x; 1.2071x over previous
import jax
import jax.numpy as jnp
from jax import lax
from jax.experimental import pallas as pl
from jax.experimental.pallas import tpu as pltpu

_T = 84
_EPS = 1e-5
_F32 = jnp.float32
_BF16 = jnp.bfloat16


def _dot(a, b):
    return jnp.dot(a, b, preferred_element_type=_F32)


def _dotr(a, b):
    return lax.dot_general(a, b, (((1,), (1,)), ((), ())),
                           preferred_element_type=_F32)


def _tree_sum(terms):
    while len(terms) > 1:
        nxt = [terms[i] + terms[i + 1] for i in range(0, len(terms) - 1, 2)]
        if len(terms) % 2:
            nxt.append(terms[-1])
        terms = nxt
    return terms[0]


_NR = 4
_TOP = 24
_STR = 112
_W = (_NR - 1) * _STR + 84
_LTOT = _TOP + _W + 20


def _dw_relu_st(xs, wdt_ref, bdt_ref, *, K, dil):
    C = xs.shape[1]
    pad = (K - 1) // 2 * dil
    wdt = wdt_ref[...]
    terms = [jnp.broadcast_to(bdt_ref[...], (_W, C))]
    for k in range(K):
        o = _TOP + k * dil - pad
        terms.append(wdt[k:k + 1, :] * xs[o:o + _W, :])
    return jnp.maximum(_tree_sum(terms), 0.0)


def _stack_pad(rows_):
    C = rows_[0].shape[1]
    gap = jnp.zeros((28, C), _F32)
    parts = [jnp.zeros((_TOP, C), _F32)]
    for r, x in enumerate(rows_):
        parts.append(x)
        parts.append(gap if r < _NR - 1 else jnp.zeros((20, C), _F32))
    return jnp.concatenate(parts, axis=0)


def _repad(z):
    C = z.shape[1]
    return jnp.concatenate(
        [jnp.zeros((_TOP, C), _F32), z, jnp.zeros((20, C), _F32)], axis=0)


def _split3(r):
    h0 = r.astype(_BF16).astype(_F32)
    r1 = r - h0
    h1 = r1.astype(_BF16).astype(_F32)
    return jnp.concatenate([h0, h1, r1 - h1], axis=0)


def _ln_pair(y, gt_ref, bt_ref, tsr_ref, scat_ref, s0t_ref, s1t_ref):
    C = y.shape[1]
    ys = [y[r * _STR:r * _STR + 84, :] for r in range(_NR)]
    t = lax.broadcasted_iota(jnp.int32, (84, C), 0)
    first = t < tsr_ref[...]
    inv_c = _F32(1.0 / C)
    scat = scat_ref[...]
    s0t = s0t_ref[...]
    s1t = s1t_ref[...]
    rsum = lambda m, i: jnp.sum(m[3 * i:3 * i + 3, :], axis=0, keepdims=True)

    def stat_rows(r):
        a0 = jnp.sum(jnp.where(first, r, 0.0), axis=0, keepdims=True)
        a1 = jnp.sum(r, axis=0, keepdims=True) - a0
        return jnp.concatenate([_split3(a0), _split3(a1)], axis=1)

    st = _dot(jnp.concatenate([stat_rows(yr) for yr in ys], axis=0), scat)
    cmu = [rsum(st, r) * inv_c for r in range(_NR)]
    cpair = jnp.concatenate([_split3(c) for c in cmu], axis=0)
    X0 = _dot(cpair, s0t)
    X1 = _dot(cpair, s1t)

    def var_rows(r, i):
        d0 = jnp.where(first, r - rsum(X0, i), 0.0)
        d1 = jnp.where(first, 0.0, r - rsum(X1, i))
        q0 = jnp.sum(d0 * d0, axis=0, keepdims=True)
        q1 = jnp.sum(d1 * d1, axis=0, keepdims=True)
        return jnp.concatenate([_split3(q0), _split3(q1)], axis=1)

    sv = _dot(jnp.concatenate([var_rows(yr, r) for r, yr in enumerate(ys)],
                              axis=0), scat)
    cvar = [rsum(sv, r) * inv_c for r in range(_NR)]
    bq = lambda v: v.astype(_BF16).astype(_F32)
    mv = jnp.concatenate(
        [bq(v) for r in range(_NR) for v in (cmu[r], cvar[r])], axis=0)
    Q0 = _dot(mv, s0t)
    Q1 = _dot(mv, s1t)
    g = gt_ref[...]
    b = bt_ref[...]
    gap = jnp.zeros((28, C), _F32)
    parts = []
    for r in range(_NR):
        mu_g = jnp.where(first, Q0[2 * r:2 * r + 1, :], Q1[2 * r:2 * r + 1, :])
        var_g = jnp.where(first, Q0[2 * r + 1:2 * r + 2, :],
                          Q1[2 * r + 1:2 * r + 2, :])
        parts.append((ys[r] - mu_g) * lax.rsqrt(var_g + _EPS) * g + b)
        if r < _NR - 1:
            parts.append(gap)
    return jnp.concatenate(parts, axis=0)


def _pair_forward(feat_ref, pv_ref,
                  wd1, bd1, wp1, bq1, g1, bb1, ts1, sc1, s0t1, s1t1,
                  wd2, bd2, wp2, bq2, g2, bb2, ts2, sc2, s0t2, s1t2,
                  wd3, bd3, wp3, bq3, g3, bb3, ts3, sc3, s0t3, s1t3,
                  wdp, bdp, wpp, bqp, gp, bbp, tsp, scp, s0tp, s1tp,
                  w1a, w1b, fb1, fw2, fb2, fw3, fb3):
    xs = _stack_pad([feat_ref[r] for r in range(_NR)])
    h = _dw_relu_st(xs, wd1, bd1, K=11, dil=1)
    y1 = jnp.maximum(_dot(h, wp1[...]) + bq1[...], 0.0)
    z1 = _ln_pair(y1, g1, bb1, ts1, sc1, s0t1, s1t1)
    h = _dw_relu_st(_repad(z1), wd2, bd2, K=11, dil=2)
    y2 = jnp.maximum(_dot(h, wp2[...]) + bq2[...], 0.0)
    z2 = _ln_pair(y2, g2, bb2, ts2, sc2, s0t2, s1t2)
    h = _dw_relu_st(_repad(z2), wd3, bd3, K=11, dil=4)
    y3 = jnp.maximum(_dot(h, wp3[...]) + bq3[...], 0.0)
    z3 = _ln_pair(y3, g3, bb3, ts3, sc3, s0t3, s1t3)

    xsp = _stack_pad([pv_ref[r] for r in range(_NR)])
    hp = _dw_relu_st(xsp, wdp, bdp, K=11, dil=1)
    yp = jnp.maximum(_dot(hp, wpp[...]) + bqp[...], 0.0)
    zp = _ln_pair(yp, gp, bbp, tsp, scp, s0tp, s1tp)

    mrow = lambda z: jnp.concatenate(
        [jnp.mean(z[r * _STR:r * _STR + 84, :], axis=0, keepdims=True)
         for r in range(_NR)], axis=0)
    p3 = mrow(z3)
    pp = mrow(zp)
    h1 = _dotr(p3, w1a[...]) + _dotr(pp, w1b[...]) + fb1[...]
    h2 = _dotr(h1, fw2[...]) + fb2[...]
    prod = (h2.astype(_BF16).astype(_F32)
            * fw3[...].astype(_BF16).astype(_F32))
    return jnp.sum(prod, axis=1, keepdims=True) + fb3[...]


def _fused_kernel(feat_ref, pv_ref, *args):
    wargs, o_ref = args[:-1], args[-1]
    o_ref[...] = _pair_forward(feat_ref, pv_ref, *wargs)


def _w2d(shape):
    n = len(shape)
    return pl.BlockSpec(tuple(shape), lambda i, n=n: (0,) * n)


def _ln_pack(g_grid, b_grid, s0, s1, tstar):
    C = g_grid.shape[0]
    return [g_grid.T, b_grid.T, tstar.reshape(1, C),
            jnp.concatenate([s0, s1], axis=0), s0.T, s1.T]


def kernel(feat, pv,
           w1d, b1d, w1p, b1p,
           w2d, b2d, w2p, b2p,
           w3d, b3d, w3p, b3p,
           wpd, bpd, wpp, bpp,
           ln1_g_grid, ln1_b_grid, ln1_s0, ln1_s1, ln1_tstar,
           ln2_g_grid, ln2_b_grid, ln2_s0, ln2_s1, ln2_tstar,
           ln3_g_grid, ln3_b_grid, ln3_s0, ln3_s1, ln3_tstar,
           lnp_g_grid, lnp_b_grid, lnp_s0, lnp_s1, lnp_tstar,
           fc_w1a, fc_w1b, fc_b1, fc_w1s, fc_b1s,
           fc_w2, fc_b2, fc_w3, fc_b3):
    B = feat.shape[0]
    feat_t = jnp.swapaxes(feat, 1, 2)
    pvr = pv.reshape(B, _T, 16 * 54)

    row = lambda v: v.reshape(1, -1)
    operands = [
        feat_t, pvr,
        w1d.T, row(b1d), w1p.T, row(b1p),
        *_ln_pack(ln1_g_grid, ln1_b_grid, ln1_s0, ln1_s1, ln1_tstar),
        w2d.T, row(b2d), w2p.T, row(b2p),
        *_ln_pack(ln2_g_grid, ln2_b_grid, ln2_s0, ln2_s1, ln2_tstar),
        w3d.T, row(b3d), w3p.T, row(b3p),
        *_ln_pack(ln3_g_grid, ln3_b_grid, ln3_s0, ln3_s1, ln3_tstar),
        wpd.T, row(bpd), wpp.T, row(bpp),
        *_ln_pack(lnp_g_grid, lnp_b_grid, lnp_s0, lnp_s1, lnp_tstar),
        fc_w1a, fc_w1b, row(fc_b1), fc_w2, row(fc_b2), fc_w3, fc_b3,
    ]
    in_specs = (
        [pl.BlockSpec((_NR, _T, 769), lambda i: (i, 0, 0)),
         pl.BlockSpec((_NR, _T, 864), lambda i: (i, 0, 0))]
        + [_w2d(op.shape) for op in operands[2:]]
    )
    out = pl.pallas_call(
        _fused_kernel,
        out_shape=jax.ShapeDtypeStruct((B // _NR, _NR, 1), _F32),
        grid=(B // _NR,),
        in_specs=in_specs,
        out_specs=pl.BlockSpec((None, _NR, 1), lambda i: (i, 0, 0)),
        compiler_params=pltpu.CompilerParams(
            dimension_semantics=("parallel",)),
    )(*operands)
    return out.reshape(B)


# speedup vs baseline: 4.4573x; 1.0868x over previous
import jax
import jax.numpy as jnp
from jax import lax
from jax.experimental import pallas as pl
from jax.experimental.pallas import tpu as pltpu

_T = 84
_EPS = 1e-5
_F32 = jnp.float32
_BF16 = jnp.bfloat16


def _dot(a, b):
    return jnp.dot(a, b, preferred_element_type=_F32)


def _dotr(a, b):
    return lax.dot_general(a, b, (((1,), (1,)), ((), ())),
                           preferred_element_type=_F32)


def _tree_sum(terms):
    while len(terms) > 1:
        nxt = [terms[i] + terms[i + 1] for i in range(0, len(terms) - 1, 2)]
        if len(terms) % 2:
            nxt.append(terms[-1])
        terms = nxt
    return terms[0]


_NR = 8
_TOP = 24
_STR = 112
_W = (_NR - 1) * _STR + 84
_LTOT = _TOP + _W + 20


def _dw_relu_st(xs, wdt_ref, bdt_ref, *, K, dil):
    C = xs.shape[1]
    pad = (K - 1) // 2 * dil
    wdt = wdt_ref[...]
    terms = [jnp.broadcast_to(bdt_ref[...], (_W, C))]
    for k in range(K):
        o = _TOP + k * dil - pad
        terms.append(wdt[k:k + 1, :] * xs[o:o + _W, :])
    return jnp.maximum(_tree_sum(terms), 0.0)


def _stack_pad(rows_):
    C = rows_[0].shape[1]
    gap = jnp.zeros((28, C), _F32)
    parts = [jnp.zeros((_TOP, C), _F32)]
    for r, x in enumerate(rows_):
        parts.append(x)
        parts.append(gap if r < _NR - 1 else jnp.zeros((20, C), _F32))
    return jnp.concatenate(parts, axis=0)


def _repad(z):
    C = z.shape[1]
    return jnp.concatenate(
        [jnp.zeros((_TOP, C), _F32), z, jnp.zeros((20, C), _F32)], axis=0)


def _split3(r):
    h0 = r.astype(_BF16).astype(_F32)
    r1 = r - h0
    h1 = r1.astype(_BF16).astype(_F32)
    return jnp.concatenate([h0, h1, r1 - h1], axis=0)


def _ln_pair(y, gt_ref, bt_ref, tsr_ref, scat_ref, s0t_ref, s1t_ref):
    C = y.shape[1]
    ys = [y[r * _STR:r * _STR + 84, :] for r in range(_NR)]
    t = lax.broadcasted_iota(jnp.int32, (84, C), 0)
    first = t < tsr_ref[...]
    inv_c = _F32(1.0 / C)
    scat = scat_ref[...]
    s0t = s0t_ref[...]
    s1t = s1t_ref[...]
    rsum = lambda m, i: jnp.sum(m[3 * i:3 * i + 3, :], axis=0, keepdims=True)

    def stat_rows(r):
        a0 = jnp.sum(jnp.where(first, r, 0.0), axis=0, keepdims=True)
        a1 = jnp.sum(r, axis=0, keepdims=True) - a0
        return jnp.concatenate([_split3(a0), _split3(a1)], axis=1)

    st = _dot(jnp.concatenate([stat_rows(yr) for yr in ys], axis=0), scat)
    cmu = [rsum(st, r) * inv_c for r in range(_NR)]
    cpair = jnp.concatenate([_split3(c) for c in cmu], axis=0)
    X0 = _dot(cpair, s0t)
    X1 = _dot(cpair, s1t)

    def var_rows(r, i):
        d0 = jnp.where(first, r - rsum(X0, i), 0.0)
        d1 = jnp.where(first, 0.0, r - rsum(X1, i))
        q0 = jnp.sum(d0 * d0, axis=0, keepdims=True)
        q1 = jnp.sum(d1 * d1, axis=0, keepdims=True)
        return jnp.concatenate([_split3(q0), _split3(q1)], axis=1)

    sv = _dot(jnp.concatenate([var_rows(yr, r) for r, yr in enumerate(ys)],
                              axis=0), scat)
    cvar = [rsum(sv, r) * inv_c for r in range(_NR)]
    bq = lambda v: v.astype(_BF16).astype(_F32)
    mv = jnp.concatenate(
        [bq(v) for r in range(_NR) for v in (cmu[r], cvar[r])], axis=0)
    Q0 = _dot(mv, s0t)
    Q1 = _dot(mv, s1t)
    g = gt_ref[...]
    b = bt_ref[...]
    gap = jnp.zeros((28, C), _F32)
    parts = []
    for r in range(_NR):
        mu_g = jnp.where(first, Q0[2 * r:2 * r + 1, :], Q1[2 * r:2 * r + 1, :])
        var_g = jnp.where(first, Q0[2 * r + 1:2 * r + 2, :],
                          Q1[2 * r + 1:2 * r + 2, :])
        parts.append((ys[r] - mu_g) * lax.rsqrt(var_g + _EPS) * g + b)
        if r < _NR - 1:
            parts.append(gap)
    return jnp.concatenate(parts, axis=0)


def _pair_forward(feat_ref, pv_ref,
                  wd1, bd1, wp1, bq1, g1, bb1, ts1, sc1, s0t1, s1t1,
                  wd2, bd2, wp2, bq2, g2, bb2, ts2, sc2, s0t2, s1t2,
                  wd3, bd3, wp3, bq3, g3, bb3, ts3, sc3, s0t3, s1t3,
                  wdp, bdp, wpp, bqp, gp, bbp, tsp, scp, s0tp, s1tp,
                  w1a, w1b, fb1, fw2, fb2, fw3, fb3):
    xs = _stack_pad([feat_ref[r] for r in range(_NR)])
    h = _dw_relu_st(xs, wd1, bd1, K=11, dil=1)
    y1 = jnp.maximum(_dot(h, wp1[...]) + bq1[...], 0.0)
    z1 = _ln_pair(y1, g1, bb1, ts1, sc1, s0t1, s1t1)
    h = _dw_relu_st(_repad(z1), wd2, bd2, K=11, dil=2)
    y2 = jnp.maximum(_dot(h, wp2[...]) + bq2[...], 0.0)
    z2 = _ln_pair(y2, g2, bb2, ts2, sc2, s0t2, s1t2)
    h = _dw_relu_st(_repad(z2), wd3, bd3, K=11, dil=4)
    y3 = jnp.maximum(_dot(h, wp3[...]) + bq3[...], 0.0)
    z3 = _ln_pair(y3, g3, bb3, ts3, sc3, s0t3, s1t3)

    xsp = _stack_pad([pv_ref[r] for r in range(_NR)])
    hp = _dw_relu_st(xsp, wdp, bdp, K=11, dil=1)
    yp = jnp.maximum(_dot(hp, wpp[...]) + bqp[...], 0.0)
    zp = _ln_pair(yp, gp, bbp, tsp, scp, s0tp, s1tp)

    mrow = lambda z: jnp.concatenate(
        [jnp.mean(z[r * _STR:r * _STR + 84, :], axis=0, keepdims=True)
         for r in range(_NR)], axis=0)
    p3 = mrow(z3)
    pp = mrow(zp)
    h1 = _dotr(p3, w1a[...]) + _dotr(pp, w1b[...]) + fb1[...]
    h2 = _dotr(h1, fw2[...]) + fb2[...]
    prod = (h2.astype(_BF16).astype(_F32)
            * fw3[...].astype(_BF16).astype(_F32))
    return jnp.sum(prod, axis=1, keepdims=True) + fb3[...]


def _fused_kernel(feat_ref, pv_ref, *args):
    wargs, o_ref = args[:-1], args[-1]
    o_ref[...] = _pair_forward(feat_ref, pv_ref, *wargs)


def _w2d(shape):
    n = len(shape)
    return pl.BlockSpec(tuple(shape), lambda i, n=n: (0,) * n)


def _ln_pack(g_grid, b_grid, s0, s1, tstar):
    C = g_grid.shape[0]
    return [g_grid.T, b_grid.T, tstar.reshape(1, C),
            jnp.concatenate([s0, s1], axis=0), s0.T, s1.T]


def kernel(feat, pv,
           w1d, b1d, w1p, b1p,
           w2d, b2d, w2p, b2p,
           w3d, b3d, w3p, b3p,
           wpd, bpd, wpp, bpp,
           ln1_g_grid, ln1_b_grid, ln1_s0, ln1_s1, ln1_tstar,
           ln2_g_grid, ln2_b_grid, ln2_s0, ln2_s1, ln2_tstar,
           ln3_g_grid, ln3_b_grid, ln3_s0, ln3_s1, ln3_tstar,
           lnp_g_grid, lnp_b_grid, lnp_s0, lnp_s1, lnp_tstar,
           fc_w1a, fc_w1b, fc_b1, fc_w1s, fc_b1s,
           fc_w2, fc_b2, fc_w3, fc_b3):
    B = feat.shape[0]
    feat_t = jnp.swapaxes(feat, 1, 2)
    pvr = pv.reshape(B, _T, 16 * 54)

    row = lambda v: v.reshape(1, -1)
    operands = [
        feat_t, pvr,
        w1d.T, row(b1d), w1p.T, row(b1p),
        *_ln_pack(ln1_g_grid, ln1_b_grid, ln1_s0, ln1_s1, ln1_tstar),
        w2d.T, row(b2d), w2p.T, row(b2p),
        *_ln_pack(ln2_g_grid, ln2_b_grid, ln2_s0, ln2_s1, ln2_tstar),
        w3d.T, row(b3d), w3p.T, row(b3p),
        *_ln_pack(ln3_g_grid, ln3_b_grid, ln3_s0, ln3_s1, ln3_tstar),
        wpd.T, row(bpd), wpp.T, row(bpp),
        *_ln_pack(lnp_g_grid, lnp_b_grid, lnp_s0, lnp_s1, lnp_tstar),
        fc_w1a, fc_w1b, row(fc_b1), fc_w2, row(fc_b2), fc_w3, fc_b3,
    ]
    in_specs = (
        [pl.BlockSpec((_NR, _T, 769), lambda i: (i, 0, 0)),
         pl.BlockSpec((_NR, _T, 864), lambda i: (i, 0, 0))]
        + [_w2d(op.shape) for op in operands[2:]]
    )
    out = pl.pallas_call(
        _fused_kernel,
        out_shape=jax.ShapeDtypeStruct((B // _NR, _NR, 1), _F32),
        grid=(B // _NR,),
        in_specs=in_specs,
        out_specs=pl.BlockSpec((None, _NR, 1), lambda i: (i, 0, 0)),
        compiler_params=pltpu.CompilerParams(
            dimension_semantics=("parallel",)),
    )(*operands)
    return out.reshape(B)
